# Initial kernel scaffold; baseline (speedup 1.0000x reference)
import jax
import jax.numpy as jnp
from jax.experimental import pallas as pl


def kernel(x, edge_index, bonds, angles, dihedrals, lj_params, W_in, W_h, bw1, bb1, bw2, bb2, aw1, ab1, aw2, ab2, dw1, db1, dw2, db2, pw1, pb1, pw2, pb2):
    raise NotImplementedError("write your pallas kernel here")


# speedup vs baseline: 6.8551x; 6.8551x over previous
import functools

import jax
import jax.numpy as jnp
from jax import lax
from jax.experimental import pallas as pl
from jax.experimental.pallas import tpu as pltpu
from jax.experimental.pallas import tpu_sc as plsc

N = 10000
E = 320000
D = 128
H = 128
NB = 20000
NA = 20000
ND = 30000
DEPTH = 3

CH = 128
CH_A = 80
CH_B = 120
NUM_SC = 2
NUM_SUBCORES = 16
ROWS_PER_TILE = 624
ROWS_TAIL_OFF = ROWS_PER_TILE * NUM_SUBCORES
ROWS_TAIL = N - ROWS_TAIL_OFF

GA = 5 * NB
GB = 4 * ND

_f32 = jnp.float32


def _vector_mesh():
    return plsc.VectorSubcoreMesh(core_axis_name="core", subcore_axis_name="subcore")


def _sc_round_agg(h, src2d, dst2d, zeros):
    @functools.partial(
        pl.kernel,
        out_type=jax.ShapeDtypeStruct((NUM_SC, N, H), _f32),
        mesh=_vector_mesh(),
        scratch_types=[
            pltpu.VMEM((CH, H), _f32),
            pltpu.VMEM_SHARED((N, H), _f32),
            pltpu.SemaphoreType.DMA,
        ],
    )
    def k(h_hbm, src_hbm, dst_hbm, z_hbm, out_hbm, rows_v, agg_sh, sem):
        cid = lax.axis_index("core")
        sid = lax.axis_index("subcore")
        row0 = sid * ROWS_PER_TILE
        pltpu.async_copy(
            z_hbm.at[pl.ds(row0, ROWS_PER_TILE)],
            agg_sh.at[pl.ds(row0, ROWS_PER_TILE)],
            sem,
        ).wait()

        @pl.when(sid == NUM_SUBCORES - 1)
        def _():
            pltpu.async_copy(
                z_hbm.at[pl.ds(ROWS_TAIL_OFF, ROWS_TAIL)],
                agg_sh.at[pl.ds(ROWS_TAIL_OFF, ROWS_TAIL)],
                sem,
            ).wait()

        plsc.subcore_barrier()

        def body(src_blk, dst_blk):
            pltpu.sync_copy(h_hbm.at[src_blk.at[0]], rows_v)
            pltpu.sync_copy(rows_v, agg_sh.at[dst_blk.at[0]], add=True)

        pltpu.emit_pipeline(
            body,
            grid=(E // CH,),
            in_specs=[
                pl.BlockSpec((1, CH), lambda i: (0, i)),
                pl.BlockSpec((1, CH), lambda i: (0, i)),
            ],
            out_specs=[],
            core_axis_name=("core", "subcore"),
            dimension_semantics=(pltpu.PARALLEL,),
        )(src_hbm, dst_hbm)
        plsc.subcore_barrier()
        pltpu.async_copy(
            agg_sh.at[pl.ds(row0, ROWS_PER_TILE)],
            out_hbm.at[cid, pl.ds(row0, ROWS_PER_TILE)],
            sem,
        ).wait()

        @pl.when(sid == NUM_SUBCORES - 1)
        def _():
            pltpu.async_copy(
                agg_sh.at[pl.ds(ROWS_TAIL_OFF, ROWS_TAIL)],
                out_hbm.at[cid, pl.ds(ROWS_TAIL_OFF, ROWS_TAIL)],
                sem,
            ).wait()

    return k(h, src2d, dst2d, zeros)


def _sc_term_gather(enc, idx_a, idx_b):
    @functools.partial(
        pl.kernel,
        out_type=(
            jax.ShapeDtypeStruct((GA, H), _f32),
            jax.ShapeDtypeStruct((GB, H), _f32),
        ),
        mesh=_vector_mesh(),
    )
    def k(enc_hbm, ia_hbm, ib_hbm, oa_hbm, ob_hbm):
        def body(i_vmem, o_vmem):
            pltpu.sync_copy(enc_hbm.at[i_vmem.at[0]], o_vmem)

        for idx_hbm, out_hbm, g, ch in (
                (ia_hbm, oa_hbm, GA, CH_A), (ib_hbm, ob_hbm, GB, CH_B)):
            pltpu.emit_pipeline(
                body,
                grid=(g // ch,),
                in_specs=[pl.BlockSpec((1, ch), lambda i: (i, 0))],
                out_specs=[pl.BlockSpec((ch, H), lambda i: (i, 0))],
                core_axis_name=("core", "subcore"),
                dimension_semantics=(pltpu.PARALLEL,),
            )(idx_hbm, out_hbm)

    return k(enc, idx_a, idx_b)


_RB = 1000


def _tc_encode_init(x, w_in):
    def body(x_ref, w_ref, o_ref):
        o_ref[...] = jnp.maximum(
            jnp.dot(x_ref[...], w_ref[...], preferred_element_type=_f32), 0.0)

    return pl.pallas_call(
        body,
        grid=(N // _RB,),
        in_specs=[
            pl.BlockSpec((_RB, D), lambda i: (i, 0)),
            pl.BlockSpec((D, H), lambda i: (0, 0)),
        ],
        out_specs=pl.BlockSpec((_RB, H), lambda i: (i, 0)),
        out_shape=jax.ShapeDtypeStruct((N, H), _f32),
    )(x, w_in)


def _tc_round_update(h, parts, w_h):
    def body(h_ref, p_ref, w_ref, o_ref):
        agg = p_ref[0] + p_ref[1]
        o_ref[...] = jnp.maximum(
            h_ref[...] + jnp.dot(agg, w_ref[...], preferred_element_type=_f32),
            0.0)

    return pl.pallas_call(
        body,
        grid=(N // _RB,),
        in_specs=[
            pl.BlockSpec((_RB, H), lambda i: (i, 0)),
            pl.BlockSpec((NUM_SC, _RB, H), lambda i: (0, i, 0)),
            pl.BlockSpec((H, H), lambda i: (0, 0)),
        ],
        out_specs=pl.BlockSpec((_RB, H), lambda i: (i, 0)),
        out_shape=jax.ShapeDtypeStruct((N, H), _f32),
    )(h, parts, w_h)


def _tc_bond_mlp(ga, w1, b1, w2, b2):
    nblk = NB // _RB

    def body(g0_ref, g1_ref, w1_ref, b1_ref, w2_ref, b2_ref, o_ref):
        e = g0_ref[...] + g1_ref[...]
        t = jnp.maximum(
            jnp.dot(e, w1_ref[...], preferred_element_type=_f32) + b1_ref[...],
            0.0)
        o_ref[...] = jnp.dot(t, w2_ref[...], preferred_element_type=_f32) + b2_ref[...]

    return pl.pallas_call(
        body,
        grid=(nblk,),
        in_specs=[
            pl.BlockSpec((_RB, H), lambda i: (i, 0)),
            pl.BlockSpec((_RB, H), lambda i: (i + nblk, 0)),
            pl.BlockSpec((H, H), lambda i: (0, 0)),
            pl.BlockSpec((1, H), lambda i: (0, 0)),
            pl.BlockSpec((H, 2), lambda i: (0, 0)),
            pl.BlockSpec((1, 2), lambda i: (0, 0)),
        ],
        out_specs=pl.BlockSpec((_RB, 2), lambda i: (i, 0)),
        out_shape=jax.ShapeDtypeStruct((NB, 2), _f32),
    )(ga, ga, w1, b1, w2, b2)


def _tc_angle_mlp(ga, w1, b1, w2, b2):
    nblk = NA // _RB
    off = 2 * (NB // _RB)

    def body(e0_ref, e2_ref, c_ref, w1_ref, b1_ref, w2_ref, b2_ref, o_ref):
        ends = e0_ref[...] + e2_ref[...]
        t = (jnp.dot(ends, w1_ref[0:H, :], preferred_element_type=_f32)
             + jnp.dot(c_ref[...], w1_ref[H:2 * H, :], preferred_element_type=_f32)
             + b1_ref[...])
        t = jnp.maximum(t, 0.0)
        o_ref[...] = jnp.dot(t, w2_ref[...], preferred_element_type=_f32) + b2_ref[...]

    return pl.pallas_call(
        body,
        grid=(nblk,),
        in_specs=[
            pl.BlockSpec((_RB, H), lambda i: (i + off, 0)),
            pl.BlockSpec((_RB, H), lambda i: (i + off + nblk, 0)),
            pl.BlockSpec((_RB, H), lambda i: (i + off + 2 * nblk, 0)),
            pl.BlockSpec((2 * H, H), lambda i: (0, 0)),
            pl.BlockSpec((1, H), lambda i: (0, 0)),
            pl.BlockSpec((H, 2), lambda i: (0, 0)),
            pl.BlockSpec((1, 2), lambda i: (0, 0)),
        ],
        out_specs=pl.BlockSpec((_RB, 2), lambda i: (i, 0)),
        out_shape=jax.ShapeDtypeStruct((NA, 2), _f32),
    )(ga, ga, ga, w1, b1, w2, b2)


def _tc_dihedral_mlp(gd, w1, w1r, b1, w2, b2):
    def body(g_ref, w1_ref, w1r_ref, b1_ref, w2_ref, b2_ref, o_ref):
        g = g_ref[...]
        t1 = jnp.maximum(
            jnp.dot(g, w1_ref[...], preferred_element_type=_f32) + b1_ref[...], 0.0)
        t2 = jnp.maximum(
            jnp.dot(g, w1r_ref[...], preferred_element_type=_f32) + b1_ref[...], 0.0)
        o_ref[...] = (0.5 * jnp.dot(t1 + t2, w2_ref[...], preferred_element_type=_f32)
                      + b2_ref[...])

    return pl.pallas_call(
        body,
        grid=(ND // _RB,),
        in_specs=[
            pl.BlockSpec((_RB, 4 * H), lambda i: (i, 0)),
            pl.BlockSpec((4 * H, H), lambda i: (0, 0)),
            pl.BlockSpec((4 * H, H), lambda i: (0, 0)),
            pl.BlockSpec((1, H), lambda i: (0, 0)),
            pl.BlockSpec((H, 4), lambda i: (0, 0)),
            pl.BlockSpec((1, 4), lambda i: (0, 0)),
        ],
        out_specs=pl.BlockSpec((_RB, 4), lambda i: (i, 0)),
        out_shape=jax.ShapeDtypeStruct((ND, 4), _f32),
    )(gd, w1, w1r, b1, w2, b2)


def _tc_pair_mlp(enc, lj, w1, b1, w2, b2):
    def body(e_ref, lj_ref, w1_ref, b1_ref, w2_ref, b2_ref, o_ref):
        t = jnp.maximum(
            jnp.dot(e_ref[...], w1_ref[...], preferred_element_type=_f32)
            + b1_ref[...], 0.0)
        tp = jnp.dot(t, w2_ref[...], preferred_element_type=_f32) + b2_ref[...]
        o_ref[...] = jnp.concatenate([tp, lj_ref[...]], axis=1)

    return pl.pallas_call(
        body,
        grid=(N // _RB,),
        in_specs=[
            pl.BlockSpec((_RB, H), lambda i: (i, 0)),
            pl.BlockSpec((_RB, 2), lambda i: (i, 0)),
            pl.BlockSpec((H, H), lambda i: (0, 0)),
            pl.BlockSpec((1, H), lambda i: (0, 0)),
            pl.BlockSpec((H, 2), lambda i: (0, 0)),
            pl.BlockSpec((1, 2), lambda i: (0, 0)),
        ],
        out_specs=pl.BlockSpec((_RB, 4), lambda i: (i, 0)),
        out_shape=jax.ShapeDtypeStruct((N, 4), _f32),
    )(enc, lj, w1, b1, w2, b2)


def kernel(x, edge_index, bonds, angles, dihedrals, lj_params,
           W_in, W_h,
           bw1, bb1, bw2, bb2,
           aw1, ab1, aw2, ab2,
           dw1, db1, dw2, db2,
           pw1, pb1, pw2, pb2):
    src2d = edge_index[0].reshape(1, E)
    dst2d = edge_index[1].reshape(1, E)
    idx_a = jnp.concatenate(
        [bonds[:, 0], bonds[:, 1], angles[:, 0], angles[:, 2], angles[:, 1]]
    ).reshape(GA // CH_A, CH_A)
    idx_b = dihedrals.reshape(GB // CH_B, CH_B)
    zeros = jnp.zeros((N, H), _f32)
    dw1r = jnp.concatenate(
        [dw1[3 * H:4 * H], dw1[2 * H:3 * H], dw1[H:2 * H], dw1[0:H]], axis=0)

    h = _tc_encode_init(x, W_in)
    for _ in range(DEPTH):
        parts = _sc_round_agg(h, src2d, dst2d, zeros)
        h = _tc_round_update(h, parts, W_h)

    ga, gd = _sc_term_gather(h, idx_a, idx_b)
    gd = gd.reshape(ND, 4 * H)

    bond_params = _tc_bond_mlp(ga, bw1, bb1.reshape(1, H), bw2, bb2.reshape(1, 2))
    angle_params = _tc_angle_mlp(ga, aw1, ab1.reshape(1, H), aw2, ab2.reshape(1, 2))
    dihedral_params = _tc_dihedral_mlp(
        gd, dw1, dw1r, db1.reshape(1, H), dw2, db2.reshape(1, 4))
    pair_params = _tc_pair_mlp(
        h, lj_params, pw1, pb1.reshape(1, H), pw2, pb2.reshape(1, 2))
    return bond_params, angle_params, dihedral_params, pair_params
